# baseline (speedup 1.0000x reference)
import functools

import jax
import jax.numpy as jnp
from jax import lax
from jax.experimental import pallas as pl
from jax.experimental.pallas import tpu as pltpu
from jax.experimental.pallas import tpu_sc as plsc

N_ROWS = 1000000
N_COLS = 64
N_UPD = 16384

NW = 32
RB = 31248
REM = N_ROWS - NW * RB
L = 16
NVEC = N_UPD // L
CAP = 2048
FL = 64

_mesh = plsc.VectorSubcoreMesh(core_axis_name="c", subcore_axis_name="s")


@functools.partial(
    pl.kernel,
    out_type=jax.ShapeDtypeStruct((N_ROWS, N_COLS), jnp.float32),
    mesh=_mesh,
    scratch_types=[
        pltpu.VMEM((N_UPD,), jnp.int32),
        pltpu.VMEM((RB + REM,), jnp.int32),
        pltpu.VMEM((CAP + L,), jnp.int32),
        pltpu.VMEM((CAP + L,), jnp.int32),
        pltpu.SemaphoreType.DMA,
        pltpu.SemaphoreType.DMA,
    ],
    compiler_params=pltpu.CompilerParams(
        needs_layout_passes=False, use_tc_tiling_on_sc=False),
)
def _sc_scatter(data_hbm, idx_hbm, upd_hbm, out_hbm,
                idxv, tagv, winsrc, wintgt, sem_a, sem_b):
    c = lax.axis_index("c")
    s = lax.axis_index("s")
    wid = c * 16 + s
    base = wid * RB
    hi = base + RB + jnp.where(wid == NW - 1, REM, 0)

    cp = pltpu.async_copy(data_hbm.at[pl.ds(base, RB)],
                          out_hbm.at[pl.ds(base, RB)], sem_b)

    pltpu.sync_copy(idx_hbm, idxv)
    lane = lax.iota(jnp.int32, L)

    def p1(k, carry):
        iv = idxv[pl.ds(k * L, L)]
        inr = (iv >= base) & (iv < hi)
        rel = jnp.where(inr, iv - base, 0)
        plsc.store_scatter(tagv, [rel], lane + k * L, mask=inr)
        return carry

    lax.fori_loop(0, NVEC, p1, 0)

    def p2(k, carry):
        n, w0, r0 = carry
        iv = idxv[pl.ds(k * L, L)]
        inr = (iv >= base) & (iv < hi)
        rel = jnp.where(inr, iv - base, 0)
        ivec = lane + k * L
        tw = plsc.load_gather(tagv, [rel], mask=inr)
        win = inr & (tw == ivec)
        wini = win.astype(jnp.int32)
        cnt = jnp.sum(wini)
        pos = n + plsc.cumsum(wini) - 1
        plsc.store_scatter(winsrc, [jnp.where(win, pos, 0)], ivec, mask=win)
        plsc.store_scatter(wintgt, [jnp.where(win, pos, 0)], iv, mask=win)
        wmax = jnp.max(jnp.where(win, ivec, -1))
        tmax = jnp.max(jnp.where(win & (ivec == wmax), iv, -1))
        w0 = jnp.where(cnt > 0, wmax, w0)
        r0 = jnp.where(cnt > 0, tmax, r0)
        return n + cnt, w0, r0

    n, w0, r0 = lax.fori_loop(0, NVEC, p2, (0, 0, base))

    nch = (n + FL - 1) // FL
    npad = nch * FL
    w0v = jnp.full((L,), w0, jnp.int32)
    r0v = jnp.full((L,), r0, jnp.int32)

    def padb(q, carry):
        p = n + q * L + lane
        m = p < npad
        plsc.store_scatter(winsrc, [jnp.where(m, p, 0)], w0v, mask=m)
        plsc.store_scatter(wintgt, [jnp.where(m, p, 0)], r0v, mask=m)
        return carry

    lax.fori_loop(0, (npad - n + L - 1) // L, padb, 0)

    cp.wait()

    @pl.when(wid == NW - 1)
    def _copy_tail():
        pltpu.sync_copy(data_hbm.at[pl.ds(NW * RB, REM)],
                        out_hbm.at[pl.ds(NW * RB, REM)])

    def scb(q, carry):
        @pl.when(q < nch)
        def _do_chunk():
            descs = []
            for g in range(FL // L):
                wv = winsrc[pl.ds(q * FL + g * L, L)]
                tv = wintgt[pl.ds(q * FL + g * L, L)]
                for j in range(L):
                    descs.append(pltpu.async_copy(
                        upd_hbm.at[pl.ds(wv[j], 1)],
                        out_hbm.at[pl.ds(tv[j], 1)], sem_a))
            for d in descs:
                d.wait()
        return carry

    lax.fori_loop(0, CAP // FL, scb, 0)


def kernel(data, indices, updates):
    idx = indices.reshape(-1)
    return _sc_scatter(data, idx, updates)


# speedup vs baseline: 5.8525x; 5.8525x over previous
import functools

import jax
import jax.numpy as jnp
from jax import lax
from jax.experimental import pallas as pl
from jax.experimental.pallas import tpu as pltpu
from jax.experimental.pallas import tpu_sc as plsc

N_ROWS = 1000000
N_COLS = 64
N_UPD = 16384

NW = 32
RB = 31248
REM = N_ROWS - NW * RB
L = 16
NVEC = N_UPD // L
CAP = 2048
FL = 64
CP = 496

_mesh = plsc.VectorSubcoreMesh(core_axis_name="c", subcore_axis_name="s")


@functools.partial(
    pl.kernel,
    out_type=jax.ShapeDtypeStruct((N_ROWS, N_COLS), jnp.float32),
    mesh=_mesh,
    scratch_types=[
        pltpu.VMEM((N_UPD,), jnp.int32),
        pltpu.VMEM((RB + REM,), jnp.int32),
        pltpu.VMEM((CAP + L,), jnp.int32),
        pltpu.VMEM((CAP + L,), jnp.int32),
        pltpu.VMEM((CP, N_COLS), jnp.float32),
        pltpu.VMEM((CP, N_COLS), jnp.float32),
        pltpu.SemaphoreType.DMA,
        pltpu.SemaphoreType.DMA,
        pltpu.SemaphoreType.DMA,
        pltpu.SemaphoreType.DMA,
    ],
    compiler_params=pltpu.CompilerParams(
        needs_layout_passes=False, use_tc_tiling_on_sc=False),
)
def _sc_scatter(data_hbm, idx_hbm, upd_hbm, out_hbm,
                idxv, tagv, winsrc, wintgt, cpbuf0, cpbuf1,
                sem_a, sem_b, sem_c, sem_d):
    c = lax.axis_index("c")
    s = lax.axis_index("s")
    wid = c * 16 + s
    base = wid * RB
    hi = base + RB + jnp.where(wid == NW - 1, REM, 0)

    def copy_pair(q, carry):
        r0 = base + q * (2 * CP)
        f0 = pltpu.async_copy(data_hbm.at[pl.ds(r0, CP)], cpbuf0, sem_a)
        f1 = pltpu.async_copy(data_hbm.at[pl.ds(r0 + CP, CP)], cpbuf1, sem_b)
        f0.wait()
        p0 = pltpu.async_copy(cpbuf0, out_hbm.at[pl.ds(r0, CP)], sem_c)
        f1.wait()
        p1 = pltpu.async_copy(cpbuf1, out_hbm.at[pl.ds(r0 + CP, CP)], sem_d)
        p0.wait()
        p1.wait()
        return carry

    lax.fori_loop(0, RB // (2 * CP), copy_pair, 0)

    pltpu.sync_copy(data_hbm.at[pl.ds(base + RB - CP, CP)], cpbuf0)
    pltpu.sync_copy(cpbuf0, out_hbm.at[pl.ds(base + RB - CP, CP)])

    @pl.when(wid == NW - 1)
    def _copy_tail():
        pltpu.sync_copy(data_hbm.at[pl.ds(NW * RB, REM)],
                        cpbuf1.at[pl.ds(0, REM)])
        pltpu.sync_copy(cpbuf1.at[pl.ds(0, REM)],
                        out_hbm.at[pl.ds(NW * RB, REM)])

    pltpu.sync_copy(idx_hbm, idxv)
    lane = lax.iota(jnp.int32, L)

    def p1(k, carry):
        iv = idxv[pl.ds(k * L, L)]
        inr = (iv >= base) & (iv < hi)
        rel = jnp.where(inr, iv - base, 0)
        plsc.store_scatter(tagv, [rel], lane + k * L, mask=inr)
        return carry

    lax.fori_loop(0, NVEC, p1, 0)

    def p2(k, carry):
        n, w0, r0 = carry
        iv = idxv[pl.ds(k * L, L)]
        inr = (iv >= base) & (iv < hi)
        rel = jnp.where(inr, iv - base, 0)
        ivec = lane + k * L
        tw = plsc.load_gather(tagv, [rel], mask=inr)
        win = inr & (tw == ivec)
        wini = win.astype(jnp.int32)
        cnt = jnp.sum(wini)
        pos = n + plsc.cumsum(wini) - 1
        plsc.store_scatter(winsrc, [jnp.where(win, pos, 0)], ivec, mask=win)
        plsc.store_scatter(wintgt, [jnp.where(win, pos, 0)], iv, mask=win)
        wmax = jnp.max(jnp.where(win, ivec, -1))
        tmax = jnp.max(jnp.where(win & (ivec == wmax), iv, -1))
        w0 = jnp.where(cnt > 0, wmax, w0)
        r0 = jnp.where(cnt > 0, tmax, r0)
        return n + cnt, w0, r0

    n, w0, r0 = lax.fori_loop(0, NVEC, p2, (0, 0, base))

    nch = (n + FL - 1) // FL
    npad = nch * FL
    w0v = jnp.full((L,), w0, jnp.int32)
    r0v = jnp.full((L,), r0, jnp.int32)

    def padb(q, carry):
        p = n + q * L + lane
        m = p < npad
        plsc.store_scatter(winsrc, [jnp.where(m, p, 0)], w0v, mask=m)
        plsc.store_scatter(wintgt, [jnp.where(m, p, 0)], r0v, mask=m)
        return carry

    lax.fori_loop(0, (npad - n + L - 1) // L, padb, 0)

    def scb(q, carry):
        @pl.when(q < nch)
        def _do_chunk():
            descs = []
            for g in range(FL // L):
                wv = winsrc[pl.ds(q * FL + g * L, L)]
                tv = wintgt[pl.ds(q * FL + g * L, L)]
                for j in range(L):
                    descs.append(pltpu.async_copy(
                        upd_hbm.at[pl.ds(wv[j], 1)],
                        out_hbm.at[pl.ds(tv[j], 1)], sem_a))
            for d in descs:
                d.wait()
        return carry

    lax.fori_loop(0, CAP // FL, scb, 0)


def kernel(data, indices, updates):
    idx = indices.reshape(-1)
    return _sc_scatter(data, idx, updates)
